# Initial kernel scaffold; baseline (speedup 1.0000x reference)
import jax
import jax.numpy as jnp
from jax.experimental import pallas as pl


def kernel(mol_x, mol_edge_index, hyper_edge, W1, as1, ad1, b1, W2, as2, ad2, b2, W3, as3, ad3, b3, fc1W, fc1b, fc2W, fc2b, mol_bias, hW1, hb1, hW2, hb2):
    raise NotImplementedError("write your pallas kernel here")


# speedup vs baseline: 5.8881x; 5.8881x over previous
import functools

import jax
import jax.numpy as jnp
from jax import lax
from jax.experimental import pallas as pl
from jax.experimental.pallas import tpu as pltpu
from jax.experimental.pallas import tpu_sc as plsc

N = 10000
N_PAD = 10240
NW = 32
NPT = 313
RP_PAD = 10112
EG = N + 320000
EG_PAD = 335872
CEPT = EG_PAD // NW
CH = 256
EH = 320000
EH_PAD = EH + 64
NEG = -3e38

_mesh = lambda: plsc.VectorSubcoreMesh(core_axis_name="c", subcore_axis_name="s",
                                       num_cores=2, num_subcores=16)
_SC_PARAMS = pltpu.CompilerParams(needs_layout_passes=False)


def _iota16():
    return lax.iota(jnp.int32, 16)


def _dg(v, idx):
    return v.at[idx].get(mode="promise_in_bounds")


def _shift_down(v, s, fill):
    it = _iota16()
    g = _dg(v, jnp.maximum(it - s, 0))
    return jnp.where(it >= s, g, fill)


def _seg_scan(key, v, combine, fill):
    for s in (1, 2, 4, 8):
        vs = _shift_down(v, s, fill)
        ks = _shift_down(key, s, jnp.int32(-1))
        v = jnp.where(key == ks, combine(v, vs), v)
    return v


def _scal(ref, i):
    return ref[pl.ds(i, 16)][0]


def _run_end(key):
    it = _iota16()
    kn = _dg(key, jnp.minimum(it + 1, 15))
    return (it == 15) | (key != kn)


def _mm_att(x, W, att_flat):
    Np, K = x.shape

    def body(x_ref, w_ref, af_ref, xw_ref, a8_ref):
        xw = jnp.dot(x_ref[...], w_ref[...], preferred_element_type=jnp.float32)
        xw_ref[...] = xw
        a8_ref[...] = jnp.dot(xw, af_ref[...], preferred_element_type=jnp.float32)

    return pl.pallas_call(
        body,
        grid=(Np // 256,),
        in_specs=[pl.BlockSpec((256, K), lambda i: (i, 0)),
                  pl.BlockSpec((K, 512), lambda i: (0, 0)),
                  pl.BlockSpec((512, 8), lambda i: (0, 0))],
        out_specs=[pl.BlockSpec((256, 512), lambda i: (i, 0)),
                   pl.BlockSpec((256, 8), lambda i: (i, 0))],
        out_shape=[jax.ShapeDtypeStruct((Np, 512), jnp.float32),
                   jax.ShapeDtypeStruct((Np, 8), jnp.float32)],
    )(x, W, att_flat)


def _mm_plain(x, W):
    Np, K = x.shape
    M = W.shape[1]

    def body(x_ref, w_ref, o_ref):
        o_ref[...] = jnp.dot(x_ref[...], w_ref[...], preferred_element_type=jnp.float32)

    return pl.pallas_call(
        body,
        grid=(Np // 256,),
        in_specs=[pl.BlockSpec((256, K), lambda i: (i, 0)),
                  pl.BlockSpec((K, M), lambda i: (0, 0))],
        out_specs=pl.BlockSpec((256, M), lambda i: (i, 0)),
        out_shape=jax.ShapeDtypeStruct((Np, M), jnp.float32),
    )(x, W)


def _tc_gate(x, mol, w1, w2, bvec):
    Np = x.shape[0]

    def body(x_ref, m_ref, w1_ref, w2_ref, b_ref, o_ref):
        xv = x_ref[...]
        mv = m_ref[...]
        z = jnp.dot(xv, w1_ref[...], preferred_element_type=jnp.float32)
        z = z + jnp.dot(mv, w2_ref[...], preferred_element_type=jnp.float32)
        z = jax.nn.sigmoid(z + b_ref[...])
        o_ref[...] = z * xv + (1.0 - z) * mv

    return pl.pallas_call(
        body,
        grid=(Np // 256,),
        in_specs=[pl.BlockSpec((256, 256), lambda i: (i, 0)),
                  pl.BlockSpec((256, 256), lambda i: (i, 0)),
                  pl.BlockSpec((256, 256), lambda i: (0, 0)),
                  pl.BlockSpec((256, 256), lambda i: (0, 0)),
                  pl.BlockSpec((1, 256), lambda i: (0, 0))],
        out_specs=pl.BlockSpec((256, 256), lambda i: (i, 0)),
        out_shape=jax.ShapeDtypeStruct((Np, 256), jnp.float32),
    )(x, mol, w1, w2, bvec)


def _gat_stats_body(a8_hbm, srcs_hbm, dsts_hbm, rp_hbm,
                    alpha_hbm, stats_hbm,
                    a8v, rpv, srcv, dstv, alpha0v, alpha1v, statsv):
    it = _iota16()
    wid = lax.axis_index("s") * 2 + lax.axis_index("c")
    n0 = wid * NPT
    nodes = jnp.minimum(NPT, N - n0)
    pltpu.sync_copy(a8_hbm, a8v)
    r0 = (n0 // 8) * 8
    pltpu.sync_copy(rp_hbm.at[pl.ds(r0, 384)], rpv)
    e_lo = _scal(rpv, n0 - r0)
    e_hi = _scal(rpv, n0 - r0 + nodes)

    patt = jnp.where(it % 8 < 2, NEG, jnp.float32(0.0))

    def initb(i, c):
        statsv[pl.ds(i * 16, 16)] = patt
        return c

    lax.fori_loop(0, 160, initb, 0)

    def alpha_group(g):
        s = srcv[pl.ds(g * 16, 16)]
        d = dstv[pl.ds(g * 16, 16)]
        s8 = s * 8
        d8 = d * 8
        out = []
        for h in range(2):
            a = plsc.load_gather(a8v, [s8 + h]) + plsc.load_gather(a8v, [d8 + 2 + h])
            out.append(jnp.maximum(a, 0.2 * a))
        return d, out

    def p1(ch, c):
        base = wid * CEPT + ch * CH
        pltpu.sync_copy(srcs_hbm.at[pl.ds(base, CH)], srcv)
        pltpu.sync_copy(dsts_hbm.at[pl.ds(base, CH)], dstv)
        for g in range(CH // 16):
            _, al = alpha_group(g)
            alpha0v[pl.ds(g * 16, 16)] = al[0]
            alpha1v[pl.ds(g * 16, 16)] = al[1]
        pltpu.sync_copy(alpha0v, alpha_hbm.at[pl.ds(base, CH)])
        pltpu.sync_copy(alpha1v, alpha_hbm.at[pl.ds(EG_PAD + base, CH)])
        return c

    lax.fori_loop(0, CEPT // CH, p1, 0)

    lo8 = (e_lo // 8) * 8
    nch = (e_hi - lo8 + CH - 1) // CH

    def p2(ch, c):
        base = lo8 + ch * CH
        pltpu.sync_copy(srcs_hbm.at[pl.ds(base, CH)], srcv)
        pltpu.sync_copy(dsts_hbm.at[pl.ds(base, CH)], dstv)
        for g in range(CH // 16):
            d, al = alpha_group(g)
            eidx = base + g * 16 + it
            valid = (eidx >= e_lo) & (eidx < e_hi)
            key = jnp.where(valid, d, jnp.int32(-2))
            rend = _run_end(key) & valid
            rel8 = (d - n0) * 8
            for h in range(2):
                v = jnp.where(valid, al[h], NEG)
                v = _seg_scan(key, v, jnp.maximum, NEG)
                cur = plsc.load_gather(statsv, [rel8 + h], mask=rend)
                plsc.store_scatter(statsv, [rel8 + h], jnp.maximum(cur, v), mask=rend)
        return c

    lax.fori_loop(0, nch, p2, 0)

    def p3(ch, c):
        base = lo8 + ch * CH
        pltpu.sync_copy(srcs_hbm.at[pl.ds(base, CH)], srcv)
        pltpu.sync_copy(dsts_hbm.at[pl.ds(base, CH)], dstv)
        for g in range(CH // 16):
            d, al = alpha_group(g)
            eidx = base + g * 16 + it
            valid = (eidx >= e_lo) & (eidx < e_hi)
            key = jnp.where(valid, d, jnp.int32(-2))
            rend = _run_end(key) & valid
            relc8 = jnp.clip(d - n0, 0, nodes - 1) * 8
            rel8 = (d - n0) * 8
            for h in range(2):
                am = plsc.load_gather(statsv, [relc8 + h])
                v = jnp.exp(jnp.where(valid, al[h], NEG) - am)
                v = jnp.where(valid, v, jnp.float32(0.0))
                v = _seg_scan(key, v, lambda a, b: a + b, jnp.float32(0.0))
                plsc.addupdate_scatter(statsv, [rel8 + 2 + h], v, mask=rend)
        return c

    lax.fori_loop(0, nch, p3, 0)

    pltpu.sync_copy(statsv, stats_hbm.at[pl.ds(wid * 2560, 2560)])


def _gat_stats(a8f, srcs, dsts, rp):
    return pl.kernel(
        _gat_stats_body,
        out_type=[jax.ShapeDtypeStruct((2 * EG_PAD,), jnp.float32),
                  jax.ShapeDtypeStruct((N_PAD * 8,), jnp.float32)],
        mesh=_mesh(),
        compiler_params=_SC_PARAMS,
        scratch_types=[pltpu.VMEM((N_PAD * 8,), jnp.float32),
                       pltpu.VMEM((384,), jnp.int32),
                       pltpu.VMEM((CH,), jnp.int32),
                       pltpu.VMEM((CH,), jnp.int32),
                       pltpu.VMEM((CH,), jnp.float32),
                       pltpu.VMEM((CH,), jnp.float32),
                       pltpu.VMEM((2560,), jnp.float32)],
    )(a8f, srcs, dsts, rp)


def _gat_agg_body(relu, xw_hbm, srcs_hbm, dsts_hbm, rp_hbm, alpha_hbm, stats_hbm,
                  bias_hbm, out_hbm,
                  statsv, biasv, rpv, ring, idxv, dstv, alv, c0v, c1v, out_local, sems):
    it = _iota16()
    wid = lax.axis_index("s") * 2 + lax.axis_index("c")
    n0 = wid * NPT
    nodes = jnp.minimum(NPT, N - n0)
    r0 = (n0 // 8) * 8
    pltpu.sync_copy(rp_hbm.at[pl.ds(r0, 384)], rpv)
    pltpu.sync_copy(stats_hbm.at[pl.ds(wid * 2560, 2560)], statsv)
    pltpu.sync_copy(bias_hbm, biasv)
    e_lo = _scal(rpv, n0 - r0)
    e_hi = _scal(rpv, n0 - r0 + nodes)
    lo8 = (e_lo // 8) * 8
    nb = (e_hi - lo8 + 31) // 32

    def stage(slot, b):
        base = lo8 + b * 32
        pltpu.sync_copy(srcs_hbm.at[pl.ds(base, 32)], idxv.at[slot])
        pltpu.sync_copy(dsts_hbm.at[pl.ds(base, 32)], dstv.at[slot, pl.ds(0, 32)])
        for h in range(2):
            pltpu.sync_copy(alpha_hbm.at[pl.ds(h * EG_PAD + base, 32)], alv.at[slot, h])
        pltpu.async_copy(xw_hbm.at[idxv.at[slot]], ring.at[slot], sems.at[slot])

    @pl.when(nb > 0)
    def _():
        stage(0, 0)

    def batch(b, accs):
        slot = b % 2
        nslot = (b + 1) % 2

        @pl.when(b + 1 < nb)
        def _():
            stage(nslot, b + 1)

        pltpu.make_async_copy(xw_hbm.at[pl.ds(0, 32)], ring.at[slot], sems.at[slot]).wait()

        for g in range(2):
            dvec = dstv[slot, pl.ds(g * 16, 16)]
            eidx = lo8 + b * 32 + g * 16 + it
            valid = (eidx >= e_lo) & (eidx < e_hi)
            rel8 = jnp.clip(dvec - n0, 0, nodes - 1) * 8
            for h in range(2):
                al = alv[slot, h, pl.ds(g * 16, 16)]
                am = plsc.load_gather(statsv, [rel8 + h])
                dn = plsc.load_gather(statsv, [rel8 + 2 + h])
                coef = jnp.exp(al - am) / (dn + 1e-16)
                coef = jnp.where(valid, coef, jnp.float32(0.0))
                if h == 0:
                    c0v[pl.ds(g * 16, 16)] = coef
                else:
                    c1v[pl.ds(g * 16, 16)] = coef

        def edge(j, accs):
            e = lo8 + b * 32 + j
            valid = (e >= e_lo) & (e < e_hi)
            c0 = _scal(c0v, j)
            c1 = _scal(c1v, j)
            d = dstv[slot, pl.ds(j, 16)][0]
            dn_next = jnp.where(j == 31, dstv[nslot, pl.ds(0, 16)][0],
                                dstv[slot, pl.ds(jnp.minimum(j + 1, 31), 16)][0])
            close = valid & ((e == e_hi - 1) | (d != dn_next))
            na = []
            for v in range(32):
                c = c0 if v < 16 else c1
                na.append(accs[v] + c * ring[slot, j, pl.ds((v % 16) * 16 + (v // 16) * 256, 16)])
            rel = d - n0

            @pl.when(close)
            def _():
                for v in range(16):
                    row = 0.5 * (na[v] + na[16 + v]) + biasv[pl.ds(v * 16, 16)]
                    if relu:
                        row = jnp.maximum(row, jnp.float32(0.0))
                    out_local[pl.ds(rel * 256 + v * 16, 16)] = row

            return tuple(jnp.where(close, jnp.float32(0.0), a) for a in na)

        return lax.fori_loop(0, 32, edge, accs)

    accs0 = tuple(jnp.zeros((16,), jnp.float32) for _ in range(32))
    lax.fori_loop(0, nb, batch, accs0)
    pltpu.sync_copy(out_local, out_hbm.at[pl.ds(n0 * 256, NPT * 256)])


def _gat_agg(xw, srcs, dsts, rp, alpha, stats, bias, relu):
    body = functools.partial(_gat_agg_body, relu)
    return pl.kernel(
        body,
        out_type=jax.ShapeDtypeStruct((N_PAD * 256,), jnp.float32),
        mesh=_mesh(),
        compiler_params=_SC_PARAMS,
        scratch_types=[pltpu.VMEM((2560,), jnp.float32),
                       pltpu.VMEM((256,), jnp.float32),
                       pltpu.VMEM((384,), jnp.int32),
                       pltpu.VMEM((2, 32, 512), jnp.float32),
                       pltpu.VMEM((2, 32), jnp.int32),
                       pltpu.VMEM((2, 48), jnp.int32),
                       pltpu.VMEM((2, 2, 32), jnp.float32),
                       pltpu.VMEM((48,), jnp.float32),
                       pltpu.VMEM((48,), jnp.float32),
                       pltpu.VMEM((NPT * 256,), jnp.float32),
                       pltpu.SemaphoreType.DMA((2,))],
    )(xw, srcs, dsts, rp, alpha, stats, bias)


def _hyp_body(relu, tab_hbm, vals_hbm, keys_hbm, rp_hbm, bias_hbm, out_hbm,
              biasv, rpv, ring, idxv, keyv, out_local, sems):
    wid = lax.axis_index("s") * 2 + lax.axis_index("c")
    n0 = wid * NPT
    nodes = jnp.minimum(NPT, N - n0)
    r0 = (n0 // 8) * 8
    pltpu.sync_copy(rp_hbm.at[pl.ds(r0, 384)], rpv)
    pltpu.sync_copy(bias_hbm, biasv)
    off = n0 - r0
    e_lo = _scal(rpv, off)
    e_hi = _scal(rpv, off + nodes)
    lo8 = (e_lo // 8) * 8
    nb = (e_hi - lo8 + 63) // 64

    def prefill(i, c):
        bi = biasv[pl.ds((i % 8) * 16, 16)]
        if relu:
            bi = jnp.maximum(bi, jnp.float32(0.0))
        out_local[pl.ds(i * 16, 16)] = bi
        return c

    lax.fori_loop(0, NPT * 8, prefill, 0)

    def stage(slot, b):
        base = lo8 + b * 64
        pltpu.sync_copy(vals_hbm.at[pl.ds(base, 64)], idxv.at[slot])
        pltpu.sync_copy(keys_hbm.at[pl.ds(base, 64)], keyv.at[slot, pl.ds(0, 64)])
        pltpu.async_copy(tab_hbm.at[idxv.at[slot]], ring.at[slot], sems.at[slot])

    @pl.when(nb > 0)
    def _():
        stage(0, 0)

    def batch(b, accs):
        slot = b % 2
        nslot = (b + 1) % 2

        @pl.when(b + 1 < nb)
        def _():
            stage(nslot, b + 1)

        pltpu.make_async_copy(tab_hbm.at[pl.ds(0, 64)], ring.at[slot], sems.at[slot]).wait()

        def edge(j, accs):
            e = lo8 + b * 64 + j
            valid = (e >= e_lo) & (e < e_hi)
            d = keyv[slot, pl.ds(j, 16)][0]
            dn_next = jnp.where(j == 63, keyv[nslot, pl.ds(0, 16)][0],
                                keyv[slot, pl.ds(jnp.minimum(j + 1, 63), 16)][0])
            close = valid & ((e == e_hi - 1) | (d != dn_next))
            na = []
            for v in range(8):
                row = ring[slot, j, pl.ds(v * 16, 16)]
                na.append(accs[v] + jnp.where(valid, row, jnp.float32(0.0)))
            rel = d - n0
            relc = off + jnp.clip(rel, 0, NPT - 1)
            cnt = _scal(rpv, relc + 1) - _scal(rpv, relc)
            cntv = jnp.zeros((16,), jnp.float32) + jnp.maximum(cnt, 1).astype(jnp.float32)
            scale = 1.0 / cntv

            @pl.when(close)
            def _():
                for v in range(8):
                    row = na[v] * scale + biasv[pl.ds(v * 16, 16)]
                    if relu:
                        row = jnp.maximum(row, jnp.float32(0.0))
                    out_local[pl.ds(rel * 128 + v * 16, 16)] = row

            return tuple(jnp.where(close, jnp.float32(0.0), a) for a in na)

        return lax.fori_loop(0, 64, edge, accs)

    accs0 = tuple(jnp.zeros((16,), jnp.float32) for _ in range(8))
    lax.fori_loop(0, nb, batch, accs0)
    pltpu.sync_copy(out_local, out_hbm.at[pl.ds(n0 * 128, NPT * 128)])


def _hyp_agg(tab, vals, keys, rp, bias, relu):
    body = functools.partial(_hyp_body, relu)
    return pl.kernel(
        body,
        out_type=jax.ShapeDtypeStruct((N_PAD * 128,), jnp.float32),
        mesh=_mesh(),
        compiler_params=_SC_PARAMS,
        scratch_types=[pltpu.VMEM((128,), jnp.float32),
                       pltpu.VMEM((384,), jnp.int32),
                       pltpu.VMEM((2, 64, 128), jnp.float32),
                       pltpu.VMEM((2, 64), jnp.int32),
                       pltpu.VMEM((2, 80), jnp.int32),
                       pltpu.VMEM((NPT * 128,), jnp.float32),
                       pltpu.SemaphoreType.DMA((2,))],
    )(tab, vals, keys, rp, bias)


def _att_flat(att_s, att_d):
    af = jnp.zeros((512, 8), jnp.float32)
    for h in range(2):
        af = af.at[h * 256:(h + 1) * 256, h].set(att_s[h])
        af = af.at[h * 256:(h + 1) * 256, 2 + h].set(att_d[h])
    return af


def _pad_rows(a, n):
    return jnp.pad(a, ((0, n - a.shape[0]),) + ((0, 0),) * (a.ndim - 1))


def _sort_edges(key, val, e_pad, pad_key):
    p = jnp.argsort(key)
    ks = key[p]
    vs = val[p]
    rp = jnp.searchsorted(ks, jnp.arange(N + 1, dtype=jnp.int32)).astype(jnp.int32)
    rp = jnp.pad(rp, (0, RP_PAD - N - 1), constant_values=key.shape[0])
    ks = jnp.pad(ks, (0, e_pad - ks.shape[0]), constant_values=pad_key)
    vs = jnp.pad(vs, (0, e_pad - vs.shape[0]))
    return ks, vs, rp


def _gat_layer(x, W, att_s, att_d, b, srcs, dsts, rp, relu):
    af = _att_flat(att_s, att_d)
    xw, a8 = _mm_att(x, W, af)
    alpha, stats = _gat_stats(a8.reshape(-1), srcs, dsts, rp)
    outf = _gat_agg(xw, srcs, dsts, rp, alpha, stats, b, relu)
    return outf.reshape(N_PAD, 256)


def kernel(mol_x, mol_edge_index, hyper_edge, W1, as1, ad1, b1, W2, as2, ad2, b2,
           W3, as3, ad3, b3, fc1W, fc1b, fc2W, fc2b, mol_bias, hW1, hb1, hW2, hb2):
    e0 = mol_edge_index[0].astype(jnp.int32)
    e1 = mol_edge_index[1].astype(jnp.int32)
    ar = jnp.arange(N, dtype=jnp.int32)
    src = jnp.concatenate([e0, ar])
    dst = jnp.concatenate([e1, ar])
    dsts, srcs, rp = _sort_edges(dst, src, EG_PAD, N)
    key1, val1, rp1 = _sort_edges(e1, e0, EH_PAD, N)
    key2, val2, rp2 = _sort_edges(e0, e1, EH_PAD, N)

    xp = jnp.pad(mol_x, ((0, N_PAD - N), (0, 128 - 78)))
    W1p = jnp.pad(W1, ((0, 128 - 78), (0, 0)))

    x = _gat_layer(xp, W1p, as1, ad1, b1, srcs, dsts, rp, relu=True)
    mol = x
    gb = (fc1b + fc2b + mol_bias[0]).reshape(1, 256)
    for (Wl, asl, adl, bl, last) in [(W2, as2, ad2, b2, False), (W3, as3, ad3, b3, True)]:
        x = _gat_layer(mol, Wl, asl, adl, bl, srcs, dsts, rp, relu=not last)
        mol = _tc_gate(x, mol, fc1W, fc2W, gb)

    hy = jnp.concatenate([mol, jnp.pad(mol_x, ((0, N_PAD - N), (0, 0))),
                          jnp.zeros((N_PAD, 50), jnp.float32)], axis=1)
    hW1p = jnp.pad(hW1, ((0, 384 - 334), (0, 0)))
    z128 = jnp.zeros((128,), jnp.float32)

    xl = _mm_plain(hy, hW1p)
    he = _hyp_agg(xl, val1, key1, rp1, z128, relu=False).reshape(N_PAD, 128)
    hyv = _hyp_agg(he, val2, key2, rp2, hb1, relu=True).reshape(N_PAD, 128)
    xl = _mm_plain(hyv, hW2)
    he = _hyp_agg(xl, val1, key1, rp1, z128, relu=False).reshape(N_PAD, 128)
    hyv = _hyp_agg(he, val2, key2, rp2, hb2, relu=True).reshape(N_PAD, 128)

    return jnp.concatenate([mol[:N], hyv[:N]], axis=1)


# speedup vs baseline: 17.4282x; 2.9599x over previous
import functools

import jax
import jax.numpy as jnp
from jax import lax
from jax.experimental import pallas as pl
from jax.experimental.pallas import tpu as pltpu
from jax.experimental.pallas import tpu_sc as plsc

N = 10000
N_PAD = 10240
NW = 32
NPT = 313
RP_PAD = 10112
EG = N + 320000
EG_PAD = 335872
CEPT = EG_PAD // NW
CH = 256
EH = 320000
EH_PAD = EH + 64
NEG = -3e38

_mesh = lambda: plsc.VectorSubcoreMesh(core_axis_name="c", subcore_axis_name="s",
                                       num_cores=2, num_subcores=16)
_SC_PARAMS = pltpu.CompilerParams(needs_layout_passes=False)


def _iota16():
    return lax.iota(jnp.int32, 16)


def _dg(v, idx):
    return v.at[idx].get(mode="promise_in_bounds")


def _shift_down(v, s, fill):
    it = _iota16()
    g = _dg(v, jnp.maximum(it - s, 0))
    return jnp.where(it >= s, g, fill)


def _seg_scan(key, v, combine, fill):
    for s in (1, 2, 4, 8):
        vs = _shift_down(v, s, fill)
        ks = _shift_down(key, s, jnp.int32(-1))
        v = jnp.where(key == ks, combine(v, vs), v)
    return v


def _scal(ref, i):
    return ref[pl.ds(i, 16)][0]


def _run_end(key):
    it = _iota16()
    kn = _dg(key, jnp.minimum(it + 1, 15))
    return (it == 15) | (key != kn)


def _mm_att(x, W, att_flat):
    Np, K = x.shape

    def body(x_ref, w_ref, af_ref, xw_ref, a8_ref):
        xw = jnp.dot(x_ref[...], w_ref[...], preferred_element_type=jnp.float32)
        xw_ref[...] = xw
        a8_ref[...] = jnp.dot(xw, af_ref[...], preferred_element_type=jnp.float32)

    return pl.pallas_call(
        body,
        grid=(Np // 256,),
        in_specs=[pl.BlockSpec((256, K), lambda i: (i, 0)),
                  pl.BlockSpec((K, 512), lambda i: (0, 0)),
                  pl.BlockSpec((512, 8), lambda i: (0, 0))],
        out_specs=[pl.BlockSpec((256, 512), lambda i: (i, 0)),
                   pl.BlockSpec((256, 8), lambda i: (i, 0))],
        out_shape=[jax.ShapeDtypeStruct((Np, 512), jnp.float32),
                   jax.ShapeDtypeStruct((Np, 8), jnp.float32)],
    )(x, W, att_flat)


def _mm_plain(x, W):
    Np, K = x.shape
    M = W.shape[1]

    def body(x_ref, w_ref, o_ref):
        o_ref[...] = jnp.dot(x_ref[...], w_ref[...], preferred_element_type=jnp.float32)

    return pl.pallas_call(
        body,
        grid=(Np // 256,),
        in_specs=[pl.BlockSpec((256, K), lambda i: (i, 0)),
                  pl.BlockSpec((K, M), lambda i: (0, 0))],
        out_specs=pl.BlockSpec((256, M), lambda i: (i, 0)),
        out_shape=jax.ShapeDtypeStruct((Np, M), jnp.float32),
    )(x, W)


def _tc_gate(x, mol, w1, w2, bvec):
    Np = x.shape[0]

    def body(x_ref, m_ref, w1_ref, w2_ref, b_ref, o_ref):
        xv = x_ref[...]
        mv = m_ref[...]
        z = jnp.dot(xv, w1_ref[...], preferred_element_type=jnp.float32)
        z = z + jnp.dot(mv, w2_ref[...], preferred_element_type=jnp.float32)
        z = jax.nn.sigmoid(z + b_ref[...])
        o_ref[...] = z * xv + (1.0 - z) * mv

    return pl.pallas_call(
        body,
        grid=(Np // 256,),
        in_specs=[pl.BlockSpec((256, 256), lambda i: (i, 0)),
                  pl.BlockSpec((256, 256), lambda i: (i, 0)),
                  pl.BlockSpec((256, 256), lambda i: (0, 0)),
                  pl.BlockSpec((256, 256), lambda i: (0, 0)),
                  pl.BlockSpec((1, 256), lambda i: (0, 0))],
        out_specs=pl.BlockSpec((256, 256), lambda i: (i, 0)),
        out_shape=jax.ShapeDtypeStruct((Np, 256), jnp.float32),
    )(x, mol, w1, w2, bvec)


def _gat_stats_body(a8_hbm, srcs_hbm, dsts_hbm, bounds_hbm,
                    alpha_hbm, stats_hbm,
                    a8v, boundsv, srcv, dstv, alpha0v, alpha1v, statsv):
    it = _iota16()
    wid = lax.axis_index("s") * 2 + lax.axis_index("c")
    n0 = wid * NPT
    nodes = jnp.minimum(NPT, N - n0)
    pltpu.sync_copy(a8_hbm, a8v)
    pltpu.sync_copy(bounds_hbm, boundsv)
    e_lo = _scal(boundsv, wid)
    e_hi = _scal(boundsv, wid + 1)

    patt = jnp.where(it % 8 < 2, NEG, jnp.float32(0.0))

    def initb(i, c):
        statsv[pl.ds(i * 16, 16)] = patt
        return c

    lax.fori_loop(0, 160, initb, 0)

    def alpha_group(g):
        s = srcv[pl.ds(g * 16, 16)]
        d = dstv[pl.ds(g * 16, 16)]
        s8 = s * 8
        d8 = d * 8
        out = []
        for h in range(2):
            a = plsc.load_gather(a8v, [s8 + h]) + plsc.load_gather(a8v, [d8 + 2 + h])
            out.append(jnp.maximum(a, 0.2 * a))
        return d, out

    def p1(ch, c):
        base = wid * CEPT + ch * CH
        pltpu.sync_copy(srcs_hbm.at[pl.ds(base, CH)], srcv)
        pltpu.sync_copy(dsts_hbm.at[pl.ds(base, CH)], dstv)
        for g in range(CH // 16):
            _, al = alpha_group(g)
            alpha0v[pl.ds(g * 16, 16)] = al[0]
            alpha1v[pl.ds(g * 16, 16)] = al[1]
        pltpu.sync_copy(alpha0v, alpha_hbm.at[pl.ds(base, CH)])
        pltpu.sync_copy(alpha1v, alpha_hbm.at[pl.ds(EG_PAD + base, CH)])
        return c

    lax.fori_loop(0, CEPT // CH, p1, 0)

    lo8 = (e_lo // 8) * 8
    nch = (e_hi - lo8 + CH - 1) // CH

    def p2(ch, c):
        base = lo8 + ch * CH
        pltpu.sync_copy(srcs_hbm.at[pl.ds(base, CH)], srcv)
        pltpu.sync_copy(dsts_hbm.at[pl.ds(base, CH)], dstv)
        for g in range(CH // 16):
            d, al = alpha_group(g)
            eidx = base + g * 16 + it
            valid = (eidx >= e_lo) & (eidx < e_hi)
            key = jnp.where(valid, d, jnp.int32(-2))
            rend = _run_end(key) & valid
            rel8 = (d - n0) * 8
            for h in range(2):
                v = jnp.where(valid, al[h], NEG)
                v = _seg_scan(key, v, jnp.maximum, NEG)
                cur = plsc.load_gather(statsv, [rel8 + h], mask=rend)
                plsc.store_scatter(statsv, [rel8 + h], jnp.maximum(cur, v), mask=rend)
        return c

    lax.fori_loop(0, nch, p2, 0)

    def p3(ch, c):
        base = lo8 + ch * CH
        pltpu.sync_copy(srcs_hbm.at[pl.ds(base, CH)], srcv)
        pltpu.sync_copy(dsts_hbm.at[pl.ds(base, CH)], dstv)
        for g in range(CH // 16):
            d, al = alpha_group(g)
            eidx = base + g * 16 + it
            valid = (eidx >= e_lo) & (eidx < e_hi)
            key = jnp.where(valid, d, jnp.int32(-2))
            rend = _run_end(key) & valid
            relc8 = jnp.clip(d - n0, 0, nodes - 1) * 8
            rel8 = (d - n0) * 8
            for h in range(2):
                am = plsc.load_gather(statsv, [relc8 + h])
                v = jnp.exp(jnp.where(valid, al[h], NEG) - am)
                v = jnp.where(valid, v, jnp.float32(0.0))
                v = _seg_scan(key, v, lambda a, b: a + b, jnp.float32(0.0))
                plsc.addupdate_scatter(statsv, [rel8 + 2 + h], v, mask=rend)
        return c

    lax.fori_loop(0, nch, p3, 0)

    pltpu.sync_copy(statsv, stats_hbm.at[pl.ds(wid * 2560, 2560)])


def _gat_stats(a8f, srcs, dsts, rp):
    return pl.kernel(
        _gat_stats_body,
        out_type=[jax.ShapeDtypeStruct((2 * EG_PAD,), jnp.float32),
                  jax.ShapeDtypeStruct((N_PAD * 8,), jnp.float32)],
        mesh=_mesh(),
        compiler_params=_SC_PARAMS,
        scratch_types=[pltpu.VMEM((N_PAD * 8,), jnp.float32),
                       pltpu.VMEM((64,), jnp.int32),
                       pltpu.VMEM((CH,), jnp.int32),
                       pltpu.VMEM((CH,), jnp.int32),
                       pltpu.VMEM((CH,), jnp.float32),
                       pltpu.VMEM((CH,), jnp.float32),
                       pltpu.VMEM((2560,), jnp.float32)],
    )(a8f, srcs, dsts, rp)


def _gat_agg_body(relu, xw_hbm, srcs_hbm, dsts_hbm, bounds_hbm, alpha_hbm, stats_hbm,
                  bias_hbm, out_hbm,
                  statsv, biasv, boundsv, ring, idxv, dstv, alv, c0v, c1v, out_local, sems):
    it = _iota16()
    wid = lax.axis_index("s") * 2 + lax.axis_index("c")
    n0 = wid * NPT
    nodes = jnp.minimum(NPT, N - n0)
    pltpu.sync_copy(bounds_hbm, boundsv)
    pltpu.sync_copy(stats_hbm.at[pl.ds(wid * 2560, 2560)], statsv)
    pltpu.sync_copy(bias_hbm, biasv)
    e_lo = _scal(boundsv, wid)
    e_hi = _scal(boundsv, wid + 1)
    lo8 = (e_lo // 8) * 8
    nb = (e_hi - lo8 + 31) // 32

    def stage(slot, b):
        base = lo8 + b * 32
        pltpu.sync_copy(srcs_hbm.at[pl.ds(base, 32)], idxv.at[slot])
        pltpu.sync_copy(dsts_hbm.at[pl.ds(base, 32)], dstv.at[slot, pl.ds(0, 32)])
        for h in range(2):
            pltpu.sync_copy(alpha_hbm.at[pl.ds(h * EG_PAD + base, 32)], alv.at[slot, h])
        pltpu.async_copy(xw_hbm.at[idxv.at[slot]], ring.at[slot], sems.at[slot])

    @pl.when(nb > 0)
    def _():
        stage(0, 0)

    def batch(b, accs):
        slot = b % 2
        nslot = (b + 1) % 2

        @pl.when(b + 1 < nb)
        def _():
            stage(nslot, b + 1)

        pltpu.make_async_copy(xw_hbm.at[pl.ds(0, 32)], ring.at[slot], sems.at[slot]).wait()

        for g in range(2):
            dvec = dstv[slot, pl.ds(g * 16, 16)]
            eidx = lo8 + b * 32 + g * 16 + it
            valid = (eidx >= e_lo) & (eidx < e_hi)
            rel8 = jnp.clip(dvec - n0, 0, nodes - 1) * 8
            for h in range(2):
                al = alv[slot, h, pl.ds(g * 16, 16)]
                am = plsc.load_gather(statsv, [rel8 + h])
                dn = plsc.load_gather(statsv, [rel8 + 2 + h])
                coef = jnp.exp(al - am) / (dn + 1e-16)
                coef = jnp.where(valid, coef, jnp.float32(0.0))
                if h == 0:
                    c0v[pl.ds(g * 16, 16)] = coef
                else:
                    c1v[pl.ds(g * 16, 16)] = coef

        def edge(j, accs):
            e = lo8 + b * 32 + j
            valid = (e >= e_lo) & (e < e_hi)
            c0 = _scal(c0v, j)
            c1 = _scal(c1v, j)
            d = dstv[slot, pl.ds(j, 16)][0]
            dn_next = jnp.where(j == 31, dstv[nslot, pl.ds(0, 16)][0],
                                dstv[slot, pl.ds(jnp.minimum(j + 1, 31), 16)][0])
            close = valid & ((e == e_hi - 1) | (d != dn_next))
            na = []
            for v in range(32):
                c = c0 if v < 16 else c1
                na.append(accs[v] + c * ring[slot, j, pl.ds((v % 16) * 16 + (v // 16) * 256, 16)])
            rel = d - n0

            @pl.when(close)
            def _():
                for v in range(16):
                    row = 0.5 * (na[v] + na[16 + v]) + biasv[pl.ds(v * 16, 16)]
                    if relu:
                        row = jnp.maximum(row, jnp.float32(0.0))
                    out_local[pl.ds(rel * 256 + v * 16, 16)] = row

            return tuple(jnp.where(close, jnp.float32(0.0), a) for a in na)

        return lax.fori_loop(0, 32, edge, accs)

    accs0 = tuple(jnp.zeros((16,), jnp.float32) for _ in range(32))
    lax.fori_loop(0, nb, batch, accs0)
    pltpu.sync_copy(out_local, out_hbm.at[pl.ds(n0 * 256, NPT * 256)])


def _gat_agg(xw, srcs, dsts, rp, alpha, stats, bias, relu):
    body = functools.partial(_gat_agg_body, relu)
    return pl.kernel(
        body,
        out_type=jax.ShapeDtypeStruct((N_PAD * 256,), jnp.float32),
        mesh=_mesh(),
        compiler_params=_SC_PARAMS,
        scratch_types=[pltpu.VMEM((2560,), jnp.float32),
                       pltpu.VMEM((256,), jnp.float32),
                       pltpu.VMEM((64,), jnp.int32),
                       pltpu.VMEM((2, 32, 512), jnp.float32),
                       pltpu.VMEM((2, 32), jnp.int32),
                       pltpu.VMEM((2, 48), jnp.int32),
                       pltpu.VMEM((2, 2, 32), jnp.float32),
                       pltpu.VMEM((48,), jnp.float32),
                       pltpu.VMEM((48,), jnp.float32),
                       pltpu.VMEM((NPT * 256,), jnp.float32),
                       pltpu.SemaphoreType.DMA((2,))],
    )(xw, srcs, dsts, rp, alpha, stats, bias)


def _hyp_body(relu, tab_hbm, vals_hbm, keys_hbm, bounds_hbm, bias_hbm, out_hbm,
              biasv, boundsv, ring, idxv, keyv, out_local, sems):
    wid = lax.axis_index("s") * 2 + lax.axis_index("c")
    n0 = wid * NPT
    nodes = jnp.minimum(NPT, N - n0)
    pltpu.sync_copy(bounds_hbm, boundsv)
    pltpu.sync_copy(bias_hbm, biasv)
    e_lo = _scal(boundsv, wid)
    e_hi = _scal(boundsv, wid + 1)
    lo8 = (e_lo // 8) * 8
    nb = (e_hi - lo8 + 63) // 64

    def prefill(i, c):
        bi = biasv[pl.ds((i % 8) * 16, 16)]
        if relu:
            bi = jnp.maximum(bi, jnp.float32(0.0))
        out_local[pl.ds(i * 16, 16)] = bi
        return c

    lax.fori_loop(0, NPT * 8, prefill, 0)

    def stage(slot, b):
        base = lo8 + b * 64
        pltpu.sync_copy(vals_hbm.at[pl.ds(base, 64)], idxv.at[slot])
        pltpu.sync_copy(keys_hbm.at[pl.ds(base, 64)], keyv.at[slot, pl.ds(0, 64)])
        pltpu.async_copy(tab_hbm.at[idxv.at[slot]], ring.at[slot], sems.at[slot])

    @pl.when(nb > 0)
    def _():
        stage(0, 0)

    def batch(b, accs):
        slot = b % 2
        nslot = (b + 1) % 2

        @pl.when(b + 1 < nb)
        def _():
            stage(nslot, b + 1)

        pltpu.make_async_copy(tab_hbm.at[pl.ds(0, 64)], ring.at[slot], sems.at[slot]).wait()

        def edge(j, carry):
            cntf, accs = carry
            e = lo8 + b * 64 + j
            valid = (e >= e_lo) & (e < e_hi)
            d = keyv[slot, pl.ds(j, 16)][0]
            dn_next = jnp.where(j == 63, keyv[nslot, pl.ds(0, 16)][0],
                                keyv[slot, pl.ds(jnp.minimum(j + 1, 63), 16)][0])
            close = valid & ((e == e_hi - 1) | (d != dn_next))
            na = []
            for v in range(8):
                row = ring[slot, j, pl.ds(v * 16, 16)]
                na.append(accs[v] + jnp.where(valid, row, jnp.float32(0.0)))
            cntf = cntf + jnp.where(valid, jnp.float32(1.0), jnp.float32(0.0))
            rel = d - n0
            scale = 1.0 / (jnp.zeros((16,), jnp.float32) + jnp.maximum(cntf, 1.0))

            @pl.when(close)
            def _():
                for v in range(8):
                    row = na[v] * scale + biasv[pl.ds(v * 16, 16)]
                    if relu:
                        row = jnp.maximum(row, jnp.float32(0.0))
                    out_local[pl.ds(rel * 128 + v * 16, 16)] = row

            return (jnp.where(close, jnp.float32(0.0), cntf),
                    tuple(jnp.where(close, jnp.float32(0.0), a) for a in na))

        return lax.fori_loop(0, 64, edge, (accs[0], accs[1]))

    accs0 = (jnp.float32(0.0), tuple(jnp.zeros((16,), jnp.float32) for _ in range(8)))
    lax.fori_loop(0, nb, batch, accs0)
    pltpu.sync_copy(out_local, out_hbm.at[pl.ds(n0 * 128, NPT * 128)])


def _hyp_agg(tab, vals, keys, rp, bias, relu):
    body = functools.partial(_hyp_body, relu)
    return pl.kernel(
        body,
        out_type=jax.ShapeDtypeStruct((N_PAD * 128,), jnp.float32),
        mesh=_mesh(),
        compiler_params=_SC_PARAMS,
        scratch_types=[pltpu.VMEM((128,), jnp.float32),
                       pltpu.VMEM((64,), jnp.int32),
                       pltpu.VMEM((2, 64, 128), jnp.float32),
                       pltpu.VMEM((2, 64), jnp.int32),
                       pltpu.VMEM((2, 80), jnp.int32),
                       pltpu.VMEM((NPT * 128,), jnp.float32),
                       pltpu.SemaphoreType.DMA((2,))],
    )(tab, vals, keys, rp, bias)


def _att_flat(att_s, att_d):
    af = jnp.zeros((512, 8), jnp.float32)
    for h in range(2):
        af = af.at[h * 256:(h + 1) * 256, h].set(att_s[h])
        af = af.at[h * 256:(h + 1) * 256, 2 + h].set(att_d[h])
    return af


def _pad_rows(a, n):
    return jnp.pad(a, ((0, n - a.shape[0]),) + ((0, 0),) * (a.ndim - 1))


def _sort_edges(key, val, e_pad, pad_key):
    p = jnp.argsort(key)
    ks = key[p]
    vs = val[p]
    q = jnp.minimum(jnp.arange(NW + 1, dtype=jnp.int32) * NPT, N)
    bounds = jnp.searchsorted(ks, q).astype(jnp.int32)
    bounds = jnp.pad(bounds, (0, 64 - NW - 1), constant_values=key.shape[0])
    ks = jnp.pad(ks, (0, e_pad - ks.shape[0]), constant_values=pad_key)
    vs = jnp.pad(vs, (0, e_pad - vs.shape[0]))
    return ks, vs, bounds


def _gat_layer(x, W, att_s, att_d, b, srcs, dsts, bnd, relu):
    af = _att_flat(att_s, att_d)
    xw, a8 = _mm_att(x, W, af)
    alpha, stats = _gat_stats(a8.reshape(-1), srcs, dsts, bnd)
    outf = _gat_agg(xw, srcs, dsts, bnd, alpha, stats, b, relu)
    return outf.reshape(N_PAD, 256)


def kernel(mol_x, mol_edge_index, hyper_edge, W1, as1, ad1, b1, W2, as2, ad2, b2,
           W3, as3, ad3, b3, fc1W, fc1b, fc2W, fc2b, mol_bias, hW1, hb1, hW2, hb2):
    e0 = mol_edge_index[0].astype(jnp.int32)
    e1 = mol_edge_index[1].astype(jnp.int32)
    ar = jnp.arange(N, dtype=jnp.int32)
    src = jnp.concatenate([e0, ar])
    dst = jnp.concatenate([e1, ar])
    dsts, srcs, bnd = _sort_edges(dst, src, EG_PAD, N)
    key1, val1, bnd1 = _sort_edges(e1, e0, EH_PAD, N)
    key2, val2, bnd2 = _sort_edges(e0, e1, EH_PAD, N)

    xp = jnp.pad(mol_x, ((0, N_PAD - N), (0, 128 - 78)))
    W1p = jnp.pad(W1, ((0, 128 - 78), (0, 0)))

    x = _gat_layer(xp, W1p, as1, ad1, b1, srcs, dsts, bnd, relu=True)
    mol = x
    gb = (fc1b + fc2b + mol_bias[0]).reshape(1, 256)
    for (Wl, asl, adl, bl, last) in [(W2, as2, ad2, b2, False), (W3, as3, ad3, b3, True)]:
        x = _gat_layer(mol, Wl, asl, adl, bl, srcs, dsts, bnd, relu=not last)
        mol = _tc_gate(x, mol, fc1W, fc2W, gb)

    hy = jnp.concatenate([mol, jnp.pad(mol_x, ((0, N_PAD - N), (0, 0))),
                          jnp.zeros((N_PAD, 50), jnp.float32)], axis=1)
    hW1p = jnp.pad(hW1, ((0, 384 - 334), (0, 0)))
    z128 = jnp.zeros((128,), jnp.float32)

    xl = _mm_plain(hy, hW1p)
    he = _hyp_agg(xl, val1, key1, bnd1, z128, relu=False).reshape(N_PAD, 128)
    hyv = _hyp_agg(he, val2, key2, bnd2, hb1, relu=True).reshape(N_PAD, 128)
    xl = _mm_plain(hyv, hW2)
    he = _hyp_agg(xl, val1, key1, bnd1, z128, relu=False).reshape(N_PAD, 128)
    hyv = _hyp_agg(he, val2, key2, bnd2, hb2, relu=True).reshape(N_PAD, 128)

    return jnp.concatenate([mol[:N], hyv[:N]], axis=1)
